# Initial kernel scaffold; baseline (speedup 1.0000x reference)
import jax
import jax.numpy as jnp
from jax.experimental import pallas as pl


def kernel(x, edge_index, substring_embed, batch, W1, b1, W2, b2, Wout, bout):
    raise NotImplementedError("write your pallas kernel here")


# speedup vs baseline: 14.5588x; 14.5588x over previous
import functools

import jax
import jax.numpy as jnp
from jax import lax
from jax.experimental import pallas as pl
from jax.experimental.pallas import tpu as pltpu
from jax.experimental.pallas import tpu_sc as plsc

N = 10000
N_PAD = 10240
H = 128
IN = 128
COND = 128
NG = 64
E = 320000
NC = 2
NS = 16
NW = NC * NS
C = 128
NCH = -(-E // (NW * C))
EW_PAD = NCH * C
E_PAD = EW_PAD * NW
RPT = N_PAD // NS
DW = 16

_mesh = plsc.VectorSubcoreMesh(core_axis_name="c", subcore_axis_name="s")


@functools.partial(
    pl.kernel,
    mesh=_mesh,
    out_type=jax.ShapeDtypeStruct((NC, N_PAD), jnp.float32),
    compiler_params=pltpu.CompilerParams(needs_layout_passes=False),
    scratch_types=[
        pltpu.VMEM((C,), jnp.int32),
        pltpu.VMEM((N_PAD,), jnp.float32),
        pltpu.VMEM((RPT,), jnp.float32),
        pltpu.VMEM((RPT,), jnp.float32),
        pltpu.VMEM_SHARED((NS, N_PAD), jnp.float32),
    ],
)
def _deg_kernel(dst_hbm, out_hbm, dst_v, hist_v, tmp_v, acc_v, parts_sh):
    cid = lax.axis_index("c")
    sid = lax.axis_index("s")
    wid = sid * NC + cid
    zero16 = jnp.zeros((16,), jnp.float32)
    ones16 = jnp.ones((16,), jnp.float32)

    def zbody(j, carry):
        hist_v[pl.ds(j * 16, 16)] = zero16
        return carry

    lax.fori_loop(0, N_PAD // 16, zbody, 0)

    def chunk(i, carry):
        base = wid * EW_PAD + i * C
        pltpu.sync_copy(dst_hbm.at[pl.ds(base, C)], dst_v)
        for j in range(C // 16):
            idx = dst_v[pl.ds(j * 16, 16)]
            plsc.addupdate_scatter(hist_v, [idx], ones16)
        return carry

    lax.fori_loop(0, NCH, chunk, 0)
    pltpu.sync_copy(hist_v, parts_sh.at[sid])
    plsc.subcore_barrier()

    r0 = sid * RPT

    def czero(j, carry):
        acc_v[pl.ds(j * 16, 16)] = zero16
        return carry

    lax.fori_loop(0, RPT // 16, czero, 0)
    for r in range(NS):
        pltpu.sync_copy(parts_sh.at[r, pl.ds(r0, RPT)], tmp_v)

        def cadd(j, carry):
            s = pl.ds(j * 16, 16)
            acc_v[s] = acc_v[s] + tmp_v[s]
            return carry

        lax.fori_loop(0, RPT // 16, cadd, 0)
    pltpu.sync_copy(acc_v, out_hbm.at[cid, pl.ds(r0, RPT)])


@functools.partial(
    pl.kernel,
    mesh=_mesh,
    out_type=jax.ShapeDtypeStruct((NC, N_PAD, H), jnp.float32),
    scratch_types=[
        pltpu.VMEM((C,), jnp.int32),
        pltpu.VMEM((C,), jnp.int32),
        pltpu.VMEM((C, H), jnp.float32),
        pltpu.VMEM((16, H), jnp.float32),
        pltpu.VMEM_SHARED((N_PAD, H), jnp.float32),
        pltpu.SemaphoreType.DMA,
    ],
)
def _edge_agg(y_hbm, src_hbm, dst_hbm, out_hbm,
              src_v, dst_v, rows_v, zero_v, accum_sh, sem):
    cid = lax.axis_index("c")
    sid = lax.axis_index("s")
    wid = sid * NC + cid
    z = jnp.zeros((16,), jnp.float32)
    for r in range(16):
        for j in range(H // 16):
            zero_v[r, pl.ds(j * 16, 16)] = z
    r0 = sid * RPT
    for k in range(RPT // 16):
        pltpu.sync_copy(zero_v, accum_sh.at[pl.ds(r0 + k * 16, 16)])
    plsc.subcore_barrier()

    def chunk(i, carry):
        base = wid * EW_PAD + i * C
        pltpu.sync_copy(src_hbm.at[pl.ds(base, C)], src_v)
        pltpu.sync_copy(dst_hbm.at[pl.ds(base, C)], dst_v)
        pltpu.async_copy(y_hbm.at[src_v], rows_v, sem).wait()
        pltpu.sync_copy(rows_v, accum_sh.at[dst_v], add=True)
        return carry

    lax.fori_loop(0, NCH, chunk, 0)
    plsc.subcore_barrier()
    pltpu.sync_copy(accum_sh.at[pl.ds(r0, RPT)],
                    out_hbm.at[cid, pl.ds(r0, RPT)])


_PREC = lax.Precision.HIGHEST
RB = 1024
NRB = N_PAD // RB


def _tc_pre_body(x_ref, se_ref, batch_ref, w1_ref, deg_ref, y_ref, dinv_ref):
    deg = deg_ref[0, :, :] + deg_ref[1, :, :] + 1.0
    dinv = lax.rsqrt(deg)
    w1x = w1_ref[:IN, :]
    w1c = w1_ref[IN:, :]
    cproj = jnp.dot(se_ref[...], w1c,
                    preferred_element_type=jnp.float32, precision=_PREC)
    oh = (batch_ref[...] ==
          lax.broadcasted_iota(jnp.int32, (RB, NG), 1)).astype(jnp.float32)
    xw = (jnp.dot(x_ref[...], w1x,
                  preferred_element_type=jnp.float32, precision=_PREC)
          + jnp.dot(oh, cproj,
                    preferred_element_type=jnp.float32, precision=_PREC))
    y_ref[...] = xw * dinv
    dinv_ref[...] = dinv


def _tc_mid_body(p_ref, y1_ref, dinv_ref, w2_ref, b1_ref, y2_ref):
    dinv = dinv_ref[...]
    pre = (p_ref[0] + p_ref[1] + y1_ref[...]) * dinv + b1_ref[...]
    h1 = jnp.maximum(pre, 0.0)
    y2_ref[...] = jnp.dot(h1, w2_ref[...],
                          preferred_element_type=jnp.float32,
                          precision=_PREC) * dinv


def _tc_out_body(q_ref, y2_ref, dinv_ref, wout_ref, b2_ref, bout_ref, o_ref):
    dinv = dinv_ref[...]
    h2 = jnp.maximum((q_ref[0] + q_ref[1] + y2_ref[...]) * dinv + b2_ref[...],
                     0.0)
    o_ref[...] = jnp.dot(h2, wout_ref[...],
                         preferred_element_type=jnp.float32,
                         precision=_PREC) + bout_ref[...]


def kernel(x, edge_index, substring_embed, batch, W1, b1, W2, b2, Wout, bout):
    x_pad = jnp.pad(x, ((0, N_PAD - N), (0, 0)))
    batch_pad = jnp.pad(batch.astype(jnp.int32),
                        (0, N_PAD - N)).reshape(N_PAD, 1)
    src = edge_index[0].astype(jnp.int32)
    dst = edge_index[1].astype(jnp.int32)
    pad_e = E_PAD - E
    pad_src = jnp.arange(pad_e, dtype=jnp.int32) % N
    pad_dst = N + (jnp.arange(pad_e, dtype=jnp.int32) % (N_PAD - N))
    src_p = jnp.concatenate([src, pad_src])
    dst_p = jnp.concatenate([dst, pad_dst])

    deg = _deg_kernel(dst_p).reshape(NC, N_PAD, 1)

    y1, dinv = pl.pallas_call(
        _tc_pre_body,
        grid=(NRB,),
        in_specs=[
            pl.BlockSpec((RB, IN), lambda i: (i, 0)),
            pl.BlockSpec((NG, COND), lambda i: (0, 0)),
            pl.BlockSpec((RB, 1), lambda i: (i, 0)),
            pl.BlockSpec((IN + COND, H), lambda i: (0, 0)),
            pl.BlockSpec((NC, RB, 1), lambda i: (0, i, 0)),
        ],
        out_specs=[pl.BlockSpec((RB, H), lambda i: (i, 0)),
                   pl.BlockSpec((RB, 1), lambda i: (i, 0))],
        out_shape=[jax.ShapeDtypeStruct((N_PAD, H), jnp.float32),
                   jax.ShapeDtypeStruct((N_PAD, 1), jnp.float32)],
    )(x_pad, substring_embed, batch_pad, W1, deg)

    p = _edge_agg(y1, src_p, dst_p)

    y2 = pl.pallas_call(
        _tc_mid_body,
        grid=(NRB,),
        in_specs=[
            pl.BlockSpec((NC, RB, H), lambda i: (0, i, 0)),
            pl.BlockSpec((RB, H), lambda i: (i, 0)),
            pl.BlockSpec((RB, 1), lambda i: (i, 0)),
            pl.BlockSpec((H, H), lambda i: (0, 0)),
            pl.BlockSpec((1, H), lambda i: (0, 0)),
        ],
        out_specs=pl.BlockSpec((RB, H), lambda i: (i, 0)),
        out_shape=jax.ShapeDtypeStruct((N_PAD, H), jnp.float32),
    )(p, y1, dinv, W2, b1.reshape(1, H))

    q = _edge_agg(y2, src_p, dst_p)

    o = pl.pallas_call(
        _tc_out_body,
        grid=(NRB,),
        in_specs=[
            pl.BlockSpec((NC, RB, H), lambda i: (0, i, 0)),
            pl.BlockSpec((RB, H), lambda i: (i, 0)),
            pl.BlockSpec((RB, 1), lambda i: (i, 0)),
            pl.BlockSpec((H, 1), lambda i: (0, 0)),
            pl.BlockSpec((1, H), lambda i: (0, 0)),
            pl.BlockSpec((1, 1), lambda i: (0, 0)),
        ],
        out_specs=pl.BlockSpec((RB, 1), lambda i: (i, 0)),
        out_shape=jax.ShapeDtypeStruct((N_PAD, 1), jnp.float32),
    )(q, y2, dinv, Wout, b2.reshape(1, H), bout.reshape(1, 1))

    return o[:N, 0]


# speedup vs baseline: 24.6550x; 1.6935x over previous
import functools

import jax
import jax.numpy as jnp
from jax import lax
from jax.experimental import pallas as pl
from jax.experimental.pallas import tpu as pltpu
from jax.experimental.pallas import tpu_sc as plsc

N = 10000
N_PAD = 10240
H = 128
IN = 128
COND = 128
NG = 64
E = 320000
NC = 2
NS = 16
NW = NC * NS
C = 128
NCH = 80
E_PAD = NW * NCH * C
RPT = N_PAD // NS

_mesh = plsc.VectorSubcoreMesh(core_axis_name="c", subcore_axis_name="s")


@functools.partial(
    pl.kernel,
    mesh=_mesh,
    out_type=jax.ShapeDtypeStruct((NC, N_PAD), jnp.float32),
    compiler_params=pltpu.CompilerParams(needs_layout_passes=False),
    scratch_types=[
        pltpu.VMEM((NCH, C), jnp.int32),
        pltpu.VMEM((N_PAD,), jnp.float32),
        pltpu.VMEM((RPT,), jnp.float32),
        pltpu.VMEM((RPT,), jnp.float32),
        pltpu.VMEM_SHARED((NS, N_PAD), jnp.float32),
    ],
)
def _deg_kernel(dst_hbm, out_hbm, dst_all, hist_v, tmp_v, acc_v, parts_sh):
    cid = lax.axis_index("c")
    sid = lax.axis_index("s")
    wid = sid * NC + cid
    zero16 = jnp.zeros((16,), jnp.float32)
    ones16 = jnp.ones((16,), jnp.float32)

    pltpu.sync_copy(dst_hbm.at[wid], dst_all)

    def zbody(j, carry):
        hist_v[pl.ds(j * 16, 16)] = zero16
        return carry

    lax.fori_loop(0, N_PAD // 16, zbody, 0)

    def chunk(i, carry):
        for j in range(C // 16):
            idx = dst_all[i, pl.ds(j * 16, 16)]
            plsc.addupdate_scatter(hist_v, [idx], ones16)
        return carry

    lax.fori_loop(0, NCH, chunk, 0)
    pltpu.sync_copy(hist_v, parts_sh.at[sid])
    plsc.subcore_barrier()

    r0 = sid * RPT

    def czero(j, carry):
        acc_v[pl.ds(j * 16, 16)] = zero16
        return carry

    lax.fori_loop(0, RPT // 16, czero, 0)
    for r in range(NS):
        pltpu.sync_copy(parts_sh.at[r, pl.ds(r0, RPT)], tmp_v)

        def cadd(j, carry):
            s = pl.ds(j * 16, 16)
            acc_v[s] = acc_v[s] + tmp_v[s]
            return carry

        lax.fori_loop(0, RPT // 16, cadd, 0)
    pltpu.sync_copy(acc_v, out_hbm.at[cid, pl.ds(r0, RPT)])


@functools.partial(
    pl.kernel,
    mesh=_mesh,
    out_type=jax.ShapeDtypeStruct((NC, N_PAD, H), jnp.float32),
    scratch_types=[
        pltpu.VMEM((4, C), jnp.int32),
        pltpu.VMEM((NCH, C), jnp.int32),
        pltpu.VMEM((C, H), jnp.float32),
        pltpu.VMEM((C, H), jnp.float32),
        pltpu.VMEM((16, H), jnp.float32),
        pltpu.VMEM_SHARED((N_PAD, H), jnp.float32),
        pltpu.SemaphoreType.DMA,
        pltpu.SemaphoreType.DMA,
        pltpu.SemaphoreType.DMA,
        pltpu.SemaphoreType.DMA,
        pltpu.SemaphoreType.DMA,
        pltpu.SemaphoreType.DMA,
        pltpu.SemaphoreType.DMA,
        pltpu.SemaphoreType.DMA,
        pltpu.SemaphoreType.DMA,
    ],
)
def _edge_agg(y_hbm, src_hbm, dst_hbm, out_hbm,
              src4, dst_all, rows0, rows1, zero_v, accum_sh,
              sg0, sg1, ss0, ss1, si0, si1, si2, si3, sd):
    cid = lax.axis_index("c")
    sid = lax.axis_index("s")
    wid = sid * NC + cid
    cp_d = pltpu.async_copy(dst_hbm.at[wid], dst_all, sd)
    z = jnp.zeros((16,), jnp.float32)
    for r in range(16):
        for j in range(H // 16):
            zero_v[r, pl.ds(j * 16, 16)] = z
    r0 = sid * RPT
    for k in range(RPT // 16):
        pltpu.sync_copy(zero_v, accum_sh.at[pl.ds(r0 + k * 16, 16)])
    cp_d.wait()
    plsc.subcore_barrier()

    rows = (rows0, rows1)
    sg = (sg0, sg1)
    ss = (ss0, ss1)
    si = (si0, si1, si2, si3)

    def istart(i, d):
        pltpu.async_copy(src_hbm.at[wid, i], src4.at[d], si[d])

    def iwait(d):
        pltpu.make_async_copy(src_hbm.at[wid, 0], src4.at[d], si[d]).wait()

    def gather(d, b):
        pltpu.async_copy(y_hbm.at[src4.at[d]], rows[b], sg[b])

    def gwait(b):
        pltpu.make_async_copy(y_hbm.at[src4.at[0]], rows[b], sg[b]).wait()

    def scat(i, b):
        pltpu.async_copy(rows[b], accum_sh.at[dst_all.at[i]], ss[b], add=True)

    def swait(b):
        pltpu.make_async_copy(rows[b], accum_sh.at[dst_all.at[0]],
                              ss[b]).wait()

    istart(0, 0)
    istart(1, 1)
    istart(2, 2)
    iwait(0)
    gather(0, 0)
    gwait(0)
    scat(0, 0)
    iwait(1)
    gather(1, 1)
    istart(3, 3)

    def body(it, carry):
        for k in range(4):
            i = 1 + it * 4 + k
            b = (1 + k) % 2
            bp = 1 - b
            d1 = (2 + k) % 4
            gwait(b)
            scat(i, b)
            swait(bp)
            iwait(d1)
            gather(d1, bp)
            istart(i + 3, k)
        return carry

    lax.fori_loop(0, (NCH - 4) // 4, body, 0)
    gwait(1)
    scat(NCH - 3, 1)
    swait(0)
    iwait((NCH - 2) % 4)
    gather((NCH - 2) % 4, 0)
    gwait(0)
    scat(NCH - 2, 0)
    swait(1)
    iwait((NCH - 1) % 4)
    gather((NCH - 1) % 4, 1)
    gwait(1)
    scat(NCH - 1, 1)
    swait(0)
    swait(1)
    plsc.subcore_barrier()
    pltpu.sync_copy(accum_sh.at[pl.ds(r0, RPT)],
                    out_hbm.at[cid, pl.ds(r0, RPT)])


_PREC = lax.Precision.HIGHEST
RB = 1000
NRB = N // RB


def _tc_pre_body(x_ref, se_ref, batch_ref, w1_ref, deg_ref, y_ref, dinv_ref):
    deg = deg_ref[0, :, :] + deg_ref[1, :, :] + 1.0
    dinv = lax.rsqrt(deg)
    w1x = w1_ref[:IN, :]
    w1c = w1_ref[IN:, :]
    cproj = jnp.dot(se_ref[...], w1c,
                    preferred_element_type=jnp.float32, precision=_PREC)
    oh = (batch_ref[...] ==
          lax.broadcasted_iota(jnp.int32, (RB, NG), 1)).astype(jnp.float32)
    xw = (jnp.dot(x_ref[...], w1x,
                  preferred_element_type=jnp.float32, precision=_PREC)
          + jnp.dot(oh, cproj,
                    preferred_element_type=jnp.float32, precision=_PREC))
    y_ref[...] = xw * dinv
    dinv_ref[...] = dinv


def _tc_mid_body(p_ref, y1_ref, dinv_ref, w2_ref, b1_ref, y2_ref):
    dinv = dinv_ref[...]
    pre = (p_ref[0] + p_ref[1] + y1_ref[...]) * dinv + b1_ref[...]
    h1 = jnp.maximum(pre, 0.0)
    y2_ref[...] = jnp.dot(h1, w2_ref[...],
                          preferred_element_type=jnp.float32,
                          precision=_PREC) * dinv


def _tc_out_body(q_ref, y2_ref, dinv_ref, wout_ref, b2_ref, bout_ref, o_ref):
    dinv = dinv_ref[...]
    h2 = jnp.maximum((q_ref[0] + q_ref[1] + y2_ref[...]) * dinv + b2_ref[...],
                     0.0)
    o_ref[...] = jnp.dot(h2, wout_ref[...],
                         preferred_element_type=jnp.float32,
                         precision=_PREC) + bout_ref[...]


def kernel(x, edge_index, substring_embed, batch, W1, b1, W2, b2, Wout, bout):
    batch2 = batch.astype(jnp.int32).reshape(N, 1)
    src = edge_index[0].astype(jnp.int32)
    dst = edge_index[1].astype(jnp.int32)
    pad_e = E_PAD - E
    pad_src = jnp.arange(pad_e, dtype=jnp.int32) % N
    pad_dst = N + (jnp.arange(pad_e, dtype=jnp.int32) % (N_PAD - N))
    src_p = jnp.concatenate([src, pad_src]).reshape(NW, NCH, C)
    dst_p = jnp.concatenate([dst, pad_dst]).reshape(NW, NCH, C)

    deg = _deg_kernel(dst_p).reshape(NC, N_PAD, 1)

    y1, dinv = pl.pallas_call(
        _tc_pre_body,
        grid=(NRB,),
        in_specs=[
            pl.BlockSpec((RB, IN), lambda i: (i, 0)),
            pl.BlockSpec((NG, COND), lambda i: (0, 0)),
            pl.BlockSpec((RB, 1), lambda i: (i, 0)),
            pl.BlockSpec((IN + COND, H), lambda i: (0, 0)),
            pl.BlockSpec((NC, RB, 1), lambda i: (0, i, 0)),
        ],
        out_specs=[pl.BlockSpec((RB, H), lambda i: (i, 0)),
                   pl.BlockSpec((RB, 1), lambda i: (i, 0))],
        out_shape=[jax.ShapeDtypeStruct((N, H), jnp.float32),
                   jax.ShapeDtypeStruct((N, 1), jnp.float32)],
    )(x, substring_embed, batch2, W1, deg)

    p = _edge_agg(y1, src_p, dst_p)

    y2 = pl.pallas_call(
        _tc_mid_body,
        grid=(NRB,),
        in_specs=[
            pl.BlockSpec((NC, RB, H), lambda i: (0, i, 0)),
            pl.BlockSpec((RB, H), lambda i: (i, 0)),
            pl.BlockSpec((RB, 1), lambda i: (i, 0)),
            pl.BlockSpec((H, H), lambda i: (0, 0)),
            pl.BlockSpec((1, H), lambda i: (0, 0)),
        ],
        out_specs=pl.BlockSpec((RB, H), lambda i: (i, 0)),
        out_shape=jax.ShapeDtypeStruct((N, H), jnp.float32),
    )(p, y1, dinv, W2, b1.reshape(1, H))

    q = _edge_agg(y2, src_p, dst_p)

    o = pl.pallas_call(
        _tc_out_body,
        grid=(NRB,),
        in_specs=[
            pl.BlockSpec((NC, RB, H), lambda i: (0, i, 0)),
            pl.BlockSpec((RB, H), lambda i: (i, 0)),
            pl.BlockSpec((RB, 1), lambda i: (i, 0)),
            pl.BlockSpec((H, 1), lambda i: (0, 0)),
            pl.BlockSpec((1, H), lambda i: (0, 0)),
            pl.BlockSpec((1, 1), lambda i: (0, 0)),
        ],
        out_specs=pl.BlockSpec((RB, 1), lambda i: (i, 0)),
        out_shape=jax.ShapeDtypeStruct((N, 1), jnp.float32),
    )(q, y2, dinv, Wout, b2.reshape(1, H), bout.reshape(1, 1))

    return o[:, 0]


# speedup vs baseline: 25.5030x; 1.0344x over previous
import functools

import jax
import jax.numpy as jnp
from jax import lax
from jax.experimental import pallas as pl
from jax.experimental.pallas import tpu as pltpu
from jax.experimental.pallas import tpu_sc as plsc

N = 10000
N_PAD = 10240
H = 128
IN = 128
COND = 128
NG = 64
E = 320000
NC = 2
NS = 16
NW = NC * NS
C = 128
NCH = 80
E_PAD = NW * NCH * C
RPT = N_PAD // NS

_mesh = plsc.VectorSubcoreMesh(core_axis_name="c", subcore_axis_name="s")


@functools.partial(
    pl.kernel,
    mesh=_mesh,
    out_type=jax.ShapeDtypeStruct((NC, N_PAD), jnp.float32),
    compiler_params=pltpu.CompilerParams(needs_layout_passes=False),
    scratch_types=[
        pltpu.VMEM((NCH, C), jnp.int32),
        pltpu.VMEM((N_PAD,), jnp.float32),
        pltpu.VMEM((RPT,), jnp.float32),
        pltpu.VMEM((RPT,), jnp.float32),
        pltpu.VMEM_SHARED((NS, N_PAD), jnp.float32),
    ],
)
def _deg_kernel(dst_hbm, out_hbm, dst_all, hist_v, tmp_v, acc_v, parts_sh):
    cid = lax.axis_index("c")
    sid = lax.axis_index("s")
    wid = sid * NC + cid
    zero16 = jnp.zeros((16,), jnp.float32)
    ones16 = jnp.ones((16,), jnp.float32)

    pltpu.sync_copy(dst_hbm.at[wid], dst_all)

    def zbody(j, carry):
        hist_v[pl.ds(j * 16, 16)] = zero16
        return carry

    lax.fori_loop(0, N_PAD // 16, zbody, 0)

    def chunk(i, carry):
        for j in range(C // 16):
            idx = dst_all[i, pl.ds(j * 16, 16)]
            plsc.addupdate_scatter(hist_v, [idx], ones16)
        return carry

    lax.fori_loop(0, NCH, chunk, 0)
    pltpu.sync_copy(hist_v, parts_sh.at[sid])
    plsc.subcore_barrier()

    r0 = sid * RPT

    def czero(j, carry):
        acc_v[pl.ds(j * 16, 16)] = zero16
        return carry

    lax.fori_loop(0, RPT // 16, czero, 0)
    for r in range(NS):
        pltpu.sync_copy(parts_sh.at[r, pl.ds(r0, RPT)], tmp_v)

        def cadd(j, carry):
            s = pl.ds(j * 16, 16)
            acc_v[s] = acc_v[s] + tmp_v[s]
            return carry

        lax.fori_loop(0, RPT // 16, cadd, 0)
    pltpu.sync_copy(acc_v, out_hbm.at[cid, pl.ds(r0, RPT)])


@functools.partial(
    pl.kernel,
    mesh=_mesh,
    out_type=jax.ShapeDtypeStruct((NC, N_PAD, H), jnp.float32),
    scratch_types=[
        pltpu.VMEM((4, C), jnp.int32),
        pltpu.VMEM((NCH, C), jnp.int32),
        pltpu.VMEM((C, H), jnp.float32),
        pltpu.VMEM((C, H), jnp.float32),
        pltpu.VMEM((16, H), jnp.float32),
        pltpu.VMEM_SHARED((N_PAD, H), jnp.float32),
        pltpu.SemaphoreType.DMA,
        pltpu.SemaphoreType.DMA,
        pltpu.SemaphoreType.DMA,
        pltpu.SemaphoreType.DMA,
        pltpu.SemaphoreType.DMA,
        pltpu.SemaphoreType.DMA,
        pltpu.SemaphoreType.DMA,
        pltpu.SemaphoreType.DMA,
        pltpu.SemaphoreType.DMA,
    ],
)
def _edge_agg(y_hbm, src_hbm, dst_hbm, out_hbm,
              src4, dst_all, rows0, rows1, zero_v, accum_sh,
              sg0, sg1, ss0, ss1, si0, si1, si2, si3, sd):
    cid = lax.axis_index("c")
    sid = lax.axis_index("s")
    wid = sid * NC + cid
    cp_d = pltpu.async_copy(dst_hbm.at[wid], dst_all, sd)
    z = jnp.zeros((16,), jnp.float32)
    for r in range(16):
        for j in range(H // 16):
            zero_v[r, pl.ds(j * 16, 16)] = z
    r0 = sid * RPT
    for k in range(RPT // 16):
        pltpu.sync_copy(zero_v, accum_sh.at[pl.ds(r0 + k * 16, 16)])
    cp_d.wait()
    plsc.subcore_barrier()

    rows = (rows0, rows1)
    sg = (sg0, sg1)
    ss = (ss0, ss1)
    si = (si0, si1, si2, si3)

    def istart(i, d):
        pltpu.async_copy(src_hbm.at[wid, i], src4.at[d], si[d])

    def iwait(d):
        pltpu.make_async_copy(src_hbm.at[wid, 0], src4.at[d], si[d]).wait()

    def gather(d, b):
        pltpu.async_copy(y_hbm.at[src4.at[d]], rows[b], sg[b])

    def gwait(b):
        pltpu.make_async_copy(y_hbm.at[src4.at[0]], rows[b], sg[b]).wait()

    def scat(i, b):
        pltpu.async_copy(rows[b], accum_sh.at[dst_all.at[i]], ss[b], add=True)

    def swait(b):
        pltpu.make_async_copy(rows[b], accum_sh.at[dst_all.at[0]],
                              ss[b]).wait()

    istart(0, 0)
    istart(1, 1)
    istart(2, 2)
    iwait(0)
    gather(0, 0)
    gwait(0)
    scat(0, 0)
    iwait(1)
    gather(1, 1)
    istart(3, 3)

    def body(it, carry):
        for k in range(4):
            i = 1 + it * 4 + k
            b = (1 + k) % 2
            bp = 1 - b
            d1 = (2 + k) % 4
            gwait(b)
            scat(i, b)
            swait(bp)
            iwait(d1)
            gather(d1, bp)
            istart(i + 3, k)
        return carry

    lax.fori_loop(0, (NCH - 4) // 4, body, 0)
    gwait(1)
    scat(NCH - 3, 1)
    swait(0)
    iwait((NCH - 2) % 4)
    gather((NCH - 2) % 4, 0)
    gwait(0)
    scat(NCH - 2, 0)
    swait(1)
    iwait((NCH - 1) % 4)
    gather((NCH - 1) % 4, 1)
    gwait(1)
    scat(NCH - 1, 1)
    swait(0)
    swait(1)
    plsc.subcore_barrier()
    pltpu.sync_copy(accum_sh.at[pl.ds(r0, RPT)],
                    out_hbm.at[cid, pl.ds(r0, RPT)])


_PREC = lax.Precision.HIGHEST
_DPREC = lax.Precision.DEFAULT
RB = 1000
NRB = N // RB


def _tc_pre_body(x_ref, se_ref, batch_ref, w1_ref, deg_ref, y_ref, dinv_ref):
    deg = deg_ref[0, :, :] + deg_ref[1, :, :] + 1.0
    dinv = lax.rsqrt(deg)
    w1x = w1_ref[:IN, :]
    w1c = w1_ref[IN:, :]
    cproj = jnp.dot(se_ref[...], w1c,
                    preferred_element_type=jnp.float32, precision=_DPREC)
    oh = (batch_ref[...] ==
          lax.broadcasted_iota(jnp.int32, (RB, NG), 1)).astype(jnp.float32)
    xw = (jnp.dot(x_ref[...], w1x,
                  preferred_element_type=jnp.float32, precision=_DPREC)
          + jnp.dot(oh, cproj,
                    preferred_element_type=jnp.float32, precision=_PREC))
    y_ref[...] = xw * dinv
    dinv_ref[...] = dinv


def _tc_mid_body(p_ref, y1_ref, dinv_ref, w2_ref, b1_ref, y2_ref):
    dinv = dinv_ref[...]
    pre = (p_ref[0] + p_ref[1] + y1_ref[...]) * dinv + b1_ref[...]
    h1 = jnp.maximum(pre, 0.0)
    y2_ref[...] = jnp.dot(h1, w2_ref[...],
                          preferred_element_type=jnp.float32,
                          precision=_DPREC) * dinv


def _tc_out_body(q_ref, y2_ref, dinv_ref, wout_ref, b2_ref, bout_ref, o_ref):
    dinv = dinv_ref[...]
    h2 = jnp.maximum((q_ref[0] + q_ref[1] + y2_ref[...]) * dinv + b2_ref[...],
                     0.0)
    o_ref[...] = jnp.dot(h2, wout_ref[...],
                         preferred_element_type=jnp.float32,
                         precision=_DPREC) + bout_ref[...]


def kernel(x, edge_index, substring_embed, batch, W1, b1, W2, b2, Wout, bout):
    batch2 = batch.astype(jnp.int32).reshape(N, 1)
    src = edge_index[0].astype(jnp.int32)
    dst = edge_index[1].astype(jnp.int32)
    pad_e = E_PAD - E
    pad_src = jnp.arange(pad_e, dtype=jnp.int32) % N
    pad_dst = N + (jnp.arange(pad_e, dtype=jnp.int32) % (N_PAD - N))
    src_p = jnp.concatenate([src, pad_src]).reshape(NW, NCH, C)
    dst_p = jnp.concatenate([dst, pad_dst]).reshape(NW, NCH, C)

    deg = _deg_kernel(dst_p).reshape(NC, N_PAD, 1)

    y1, dinv = pl.pallas_call(
        _tc_pre_body,
        grid=(NRB,),
        in_specs=[
            pl.BlockSpec((RB, IN), lambda i: (i, 0)),
            pl.BlockSpec((NG, COND), lambda i: (0, 0)),
            pl.BlockSpec((RB, 1), lambda i: (i, 0)),
            pl.BlockSpec((IN + COND, H), lambda i: (0, 0)),
            pl.BlockSpec((NC, RB, 1), lambda i: (0, i, 0)),
        ],
        out_specs=[pl.BlockSpec((RB, H), lambda i: (i, 0)),
                   pl.BlockSpec((RB, 1), lambda i: (i, 0))],
        out_shape=[jax.ShapeDtypeStruct((N, H), jnp.float32),
                   jax.ShapeDtypeStruct((N, 1), jnp.float32)],
    )(x, substring_embed, batch2, W1, deg)

    p = _edge_agg(y1, src_p, dst_p)

    y2 = pl.pallas_call(
        _tc_mid_body,
        grid=(NRB,),
        in_specs=[
            pl.BlockSpec((NC, RB, H), lambda i: (0, i, 0)),
            pl.BlockSpec((RB, H), lambda i: (i, 0)),
            pl.BlockSpec((RB, 1), lambda i: (i, 0)),
            pl.BlockSpec((H, H), lambda i: (0, 0)),
            pl.BlockSpec((1, H), lambda i: (0, 0)),
        ],
        out_specs=pl.BlockSpec((RB, H), lambda i: (i, 0)),
        out_shape=jax.ShapeDtypeStruct((N, H), jnp.float32),
    )(p, y1, dinv, W2, b1.reshape(1, H))

    q = _edge_agg(y2, src_p, dst_p)

    o = pl.pallas_call(
        _tc_out_body,
        grid=(NRB,),
        in_specs=[
            pl.BlockSpec((NC, RB, H), lambda i: (0, i, 0)),
            pl.BlockSpec((RB, H), lambda i: (i, 0)),
            pl.BlockSpec((RB, 1), lambda i: (i, 0)),
            pl.BlockSpec((H, 1), lambda i: (0, 0)),
            pl.BlockSpec((1, H), lambda i: (0, 0)),
            pl.BlockSpec((1, 1), lambda i: (0, 0)),
        ],
        out_specs=pl.BlockSpec((RB, 1), lambda i: (i, 0)),
        out_shape=jax.ShapeDtypeStruct((N, 1), jnp.float32),
    )(q, y2, dinv, Wout, b2.reshape(1, H), bout.reshape(1, 1))

    return o[:, 0]


# speedup vs baseline: 26.7961x; 1.0507x over previous
import functools

import jax
import jax.numpy as jnp
from jax import lax
from jax.experimental import pallas as pl
from jax.experimental.pallas import tpu as pltpu
from jax.experimental.pallas import tpu_sc as plsc

N = 10000
N_PAD = 10240
H = 128
IN = 128
COND = 128
NG = 64
E = 320000
NC = 2
NS = 16
NW = NC * NS
C = 128
NCH = 80
E_PAD = NW * NCH * C
RPT = N_PAD // NS

_mesh = plsc.VectorSubcoreMesh(core_axis_name="c", subcore_axis_name="s")


@functools.partial(
    pl.kernel,
    mesh=_mesh,
    out_type=jax.ShapeDtypeStruct((NC, N_PAD), jnp.float32),
    compiler_params=pltpu.CompilerParams(needs_layout_passes=False),
    scratch_types=[
        pltpu.VMEM((NCH, C), jnp.int32),
        pltpu.VMEM((N_PAD,), jnp.float32),
        pltpu.VMEM((RPT,), jnp.float32),
        pltpu.VMEM((RPT,), jnp.float32),
        pltpu.VMEM_SHARED((NS, N_PAD), jnp.float32),
    ],
)
def _deg_kernel(dst_hbm, out_hbm, dst_all, hist_v, tmp_v, acc_v, parts_sh):
    cid = lax.axis_index("c")
    sid = lax.axis_index("s")
    wid = sid * NC + cid
    zero16 = jnp.zeros((16,), jnp.float32)
    ones16 = jnp.ones((16,), jnp.float32)

    pltpu.sync_copy(dst_hbm.at[wid], dst_all)

    def zbody(j, carry):
        hist_v[pl.ds(j * 16, 16)] = zero16
        return carry

    lax.fori_loop(0, N_PAD // 16, zbody, 0)

    def chunk(i, carry):
        for j in range(C // 16):
            idx = dst_all[i, pl.ds(j * 16, 16)]
            plsc.addupdate_scatter(hist_v, [idx], ones16)
        return carry

    lax.fori_loop(0, NCH, chunk, 0)
    pltpu.sync_copy(hist_v, parts_sh.at[sid])
    plsc.subcore_barrier()

    r0 = sid * RPT

    def czero(j, carry):
        acc_v[pl.ds(j * 16, 16)] = zero16
        return carry

    lax.fori_loop(0, RPT // 16, czero, 0)
    for r in range(NS):
        pltpu.sync_copy(parts_sh.at[r, pl.ds(r0, RPT)], tmp_v)

        def cadd(j, carry):
            s = pl.ds(j * 16, 16)
            acc_v[s] = acc_v[s] + tmp_v[s]
            return carry

        lax.fori_loop(0, RPT // 16, cadd, 0)
    pltpu.sync_copy(acc_v, out_hbm.at[cid, pl.ds(r0, RPT)])


@functools.partial(
    pl.kernel,
    mesh=_mesh,
    out_type=jax.ShapeDtypeStruct((NC, N_PAD, H), jnp.bfloat16),
    compiler_params=pltpu.CompilerParams(use_tc_tiling_on_sc=False),
    scratch_types=[
        pltpu.VMEM((4, C), jnp.int32),
        pltpu.VMEM((NCH, C), jnp.int32),
        pltpu.VMEM((C, H), jnp.bfloat16),
        pltpu.VMEM((C, H), jnp.bfloat16),
        pltpu.VMEM((16, H), jnp.bfloat16),
        pltpu.VMEM_SHARED((N_PAD, H), jnp.bfloat16),
        pltpu.SemaphoreType.DMA,
        pltpu.SemaphoreType.DMA,
        pltpu.SemaphoreType.DMA,
        pltpu.SemaphoreType.DMA,
        pltpu.SemaphoreType.DMA,
        pltpu.SemaphoreType.DMA,
        pltpu.SemaphoreType.DMA,
        pltpu.SemaphoreType.DMA,
        pltpu.SemaphoreType.DMA,
    ],
)
def _edge_agg(y_hbm, src_hbm, dst_hbm, out_hbm,
              src4, dst_all, rows0, rows1, zero_v, accum_sh,
              sg0, sg1, ss0, ss1, si0, si1, si2, si3, sd):
    cid = lax.axis_index("c")
    sid = lax.axis_index("s")
    wid = sid * NC + cid
    cp_d = pltpu.async_copy(dst_hbm.at[wid], dst_all, sd)
    z = jnp.zeros((32,), jnp.bfloat16)
    for r in range(16):
        for j in range(H // 32):
            zero_v[r, pl.ds(j * 32, 32)] = z
    r0 = sid * RPT
    for k in range(RPT // 16):
        pltpu.sync_copy(zero_v, accum_sh.at[pl.ds(r0 + k * 16, 16)])
    cp_d.wait()
    plsc.subcore_barrier()

    rows = (rows0, rows1)
    sg = (sg0, sg1)
    ss = (ss0, ss1)
    si = (si0, si1, si2, si3)

    def istart(i, d):
        pltpu.async_copy(src_hbm.at[wid, i], src4.at[d], si[d])

    def iwait(d):
        pltpu.make_async_copy(src_hbm.at[wid, 0], src4.at[d], si[d]).wait()

    def gather(d, b):
        pltpu.async_copy(y_hbm.at[src4.at[d]], rows[b], sg[b])

    def gwait(b):
        pltpu.make_async_copy(y_hbm.at[src4.at[0]], rows[b], sg[b]).wait()

    def scat(i, b):
        pltpu.async_copy(rows[b], accum_sh.at[dst_all.at[i]], ss[b], add=True)

    def swait(b):
        pltpu.make_async_copy(rows[b], accum_sh.at[dst_all.at[0]],
                              ss[b]).wait()

    istart(0, 0)
    istart(1, 1)
    istart(2, 2)
    iwait(0)
    gather(0, 0)
    gwait(0)
    scat(0, 0)
    iwait(1)
    gather(1, 1)
    istart(3, 3)

    def body(it, carry):
        for k in range(4):
            i = 1 + it * 4 + k
            b = (1 + k) % 2
            bp = 1 - b
            d1 = (2 + k) % 4
            gwait(b)
            scat(i, b)
            swait(bp)
            iwait(d1)
            gather(d1, bp)
            istart(i + 3, k)
        return carry

    lax.fori_loop(0, (NCH - 4) // 4, body, 0)
    gwait(1)
    scat(NCH - 3, 1)
    swait(0)
    iwait((NCH - 2) % 4)
    gather((NCH - 2) % 4, 0)
    gwait(0)
    scat(NCH - 2, 0)
    swait(1)
    iwait((NCH - 1) % 4)
    gather((NCH - 1) % 4, 1)
    gwait(1)
    scat(NCH - 1, 1)
    swait(0)
    swait(1)
    plsc.subcore_barrier()
    pltpu.sync_copy(accum_sh.at[pl.ds(r0, RPT)],
                    out_hbm.at[cid, pl.ds(r0, RPT)])


_PREC = lax.Precision.HIGHEST
_DPREC = lax.Precision.DEFAULT
RB = 1000
NRB = N // RB


def _tc_pre_body(x_ref, se_ref, batch_ref, w1_ref, deg_ref, y_ref, dinv_ref):
    deg = deg_ref[0, :, :] + deg_ref[1, :, :] + 1.0
    dinv = lax.rsqrt(deg)
    w1x = w1_ref[:IN, :]
    w1c = w1_ref[IN:, :]
    cproj = jnp.dot(se_ref[...], w1c,
                    preferred_element_type=jnp.float32, precision=_DPREC)
    oh = (batch_ref[...] ==
          lax.broadcasted_iota(jnp.int32, (RB, NG), 1)).astype(jnp.float32)
    xw = (jnp.dot(x_ref[...], w1x,
                  preferred_element_type=jnp.float32, precision=_DPREC)
          + jnp.dot(oh, cproj,
                    preferred_element_type=jnp.float32, precision=_PREC))
    y_ref[...] = (xw * dinv).astype(jnp.bfloat16)
    dinv_ref[...] = dinv


def _tc_mid_body(p_ref, y1_ref, dinv_ref, w2_ref, b1_ref, y2_ref):
    dinv = dinv_ref[...]
    agg = (p_ref[0].astype(jnp.float32) + p_ref[1].astype(jnp.float32)
           + y1_ref[...].astype(jnp.float32))
    pre = agg * dinv + b1_ref[...]
    h1 = jnp.maximum(pre, 0.0)
    y2_ref[...] = (jnp.dot(h1, w2_ref[...],
                           preferred_element_type=jnp.float32,
                           precision=_DPREC) * dinv).astype(jnp.bfloat16)


def _tc_out_body(q_ref, y2_ref, dinv_ref, wout_ref, b2_ref, bout_ref, o_ref):
    dinv = dinv_ref[...]
    agg = (q_ref[0].astype(jnp.float32) + q_ref[1].astype(jnp.float32)
           + y2_ref[...].astype(jnp.float32))
    h2 = jnp.maximum(agg * dinv + b2_ref[...], 0.0)
    o_ref[...] = jnp.dot(h2, wout_ref[...],
                         preferred_element_type=jnp.float32,
                         precision=_DPREC) + bout_ref[...]


def kernel(x, edge_index, substring_embed, batch, W1, b1, W2, b2, Wout, bout):
    batch2 = batch.astype(jnp.int32).reshape(N, 1)
    src = edge_index[0].astype(jnp.int32)
    dst = edge_index[1].astype(jnp.int32)
    pad_e = E_PAD - E
    pad_src = jnp.arange(pad_e, dtype=jnp.int32) % N
    pad_dst = N + (jnp.arange(pad_e, dtype=jnp.int32) % (N_PAD - N))
    src_p = jnp.concatenate([src, pad_src]).reshape(NW, NCH, C)
    dst_p = jnp.concatenate([dst, pad_dst]).reshape(NW, NCH, C)

    deg = _deg_kernel(dst_p).reshape(NC, N_PAD, 1)

    y1, dinv = pl.pallas_call(
        _tc_pre_body,
        grid=(NRB,),
        in_specs=[
            pl.BlockSpec((RB, IN), lambda i: (i, 0)),
            pl.BlockSpec((NG, COND), lambda i: (0, 0)),
            pl.BlockSpec((RB, 1), lambda i: (i, 0)),
            pl.BlockSpec((IN + COND, H), lambda i: (0, 0)),
            pl.BlockSpec((NC, RB, 1), lambda i: (0, i, 0)),
        ],
        out_specs=[pl.BlockSpec((RB, H), lambda i: (i, 0)),
                   pl.BlockSpec((RB, 1), lambda i: (i, 0))],
        out_shape=[jax.ShapeDtypeStruct((N, H), jnp.bfloat16),
                   jax.ShapeDtypeStruct((N, 1), jnp.float32)],
    )(x, substring_embed, batch2, W1, deg)

    p = _edge_agg(y1, src_p, dst_p)

    y2 = pl.pallas_call(
        _tc_mid_body,
        grid=(NRB,),
        in_specs=[
            pl.BlockSpec((NC, RB, H), lambda i: (0, i, 0)),
            pl.BlockSpec((RB, H), lambda i: (i, 0)),
            pl.BlockSpec((RB, 1), lambda i: (i, 0)),
            pl.BlockSpec((H, H), lambda i: (0, 0)),
            pl.BlockSpec((1, H), lambda i: (0, 0)),
        ],
        out_specs=pl.BlockSpec((RB, H), lambda i: (i, 0)),
        out_shape=jax.ShapeDtypeStruct((N, H), jnp.bfloat16),
    )(p, y1, dinv, W2, b1.reshape(1, H))

    q = _edge_agg(y2, src_p, dst_p)

    o = pl.pallas_call(
        _tc_out_body,
        grid=(NRB,),
        in_specs=[
            pl.BlockSpec((NC, RB, H), lambda i: (0, i, 0)),
            pl.BlockSpec((RB, H), lambda i: (i, 0)),
            pl.BlockSpec((RB, 1), lambda i: (i, 0)),
            pl.BlockSpec((H, 1), lambda i: (0, 0)),
            pl.BlockSpec((1, H), lambda i: (0, 0)),
            pl.BlockSpec((1, 1), lambda i: (0, 0)),
        ],
        out_specs=pl.BlockSpec((RB, 1), lambda i: (i, 0)),
        out_shape=jax.ShapeDtypeStruct((N, 1), jnp.float32),
    )(q, y2, dinv, Wout, b2.reshape(1, H), bout.reshape(1, 1))

    return o[:, 0]
